# Initial kernel scaffold; baseline (speedup 1.0000x reference)
import jax
import jax.numpy as jnp
from jax import lax
from jax.experimental import pallas as pl
from jax.experimental.pallas import tpu as pltpu
from jax.experimental.pallas import tpu_sc as plsc

_OLD_VOCAB = 1000000
_NEW_VOCAB = 1000
_EMBED_DIM = 64

_NUM_WORKERS = 32
_CHUNK = 512
_SUB = 128
_LANES = 16


def _body(ids_hbm, old_hbm, new_hbm, out_hbm,
          newtab_v, raw_v, idx_v, rows_v, gsem):
    n_ids = ids_hbm.shape[0]
    per_w = n_ids // _NUM_WORKERS
    n_chunks = per_w // _CHUNK
    wid = lax.axis_index("s") * 2 + lax.axis_index("c")
    base_w = wid * per_w

    pltpu.sync_copy(new_hbm, newtab_v)

    iota16 = lax.iota(jnp.int32, _LANES)

    def chunk_body(g, _):
        base = base_w + g * _CHUNK
        pltpu.sync_copy(ids_hbm.at[pl.ds(base, _CHUNK)], raw_v)

        def clip_body(i, _):
            v = raw_v[pl.ds(i * _LANES, _LANES)]
            idx_v[pl.ds(i * _LANES, _LANES)] = jnp.minimum(v, _OLD_VOCAB - 1)
            return 0
        lax.fori_loop(0, _CHUNK // _LANES, clip_body, 0)

        copies = []
        for j in range(_CHUNK // _SUB):
            copies.append(pltpu.async_copy(
                old_hbm.at[idx_v.at[pl.ds(j * _SUB, _SUB)]],
                rows_v.at[pl.ds(j * _SUB, _SUB)], gsem))
        for c in copies:
            c.wait()

        def fix_body(i, _):
            v = raw_v[pl.ds(i * _LANES, _LANES)]
            m = v >= _OLD_VOCAB

            @pl.when(jnp.max(v) >= _OLD_VOCAB)
            def _():
                nid = jnp.maximum(v - _OLD_VOCAB, 0)
                rowpos = iota16 + i * _LANES

                def d_body(d, _):
                    dvec = jnp.full((_LANES,), d, jnp.int32)
                    vals = plsc.load_gather(newtab_v, [nid, dvec], mask=m)
                    plsc.store_scatter(rows_v, [rowpos, dvec], vals, mask=m)
                    return 0
                lax.fori_loop(0, _EMBED_DIM, d_body, 0)
            return 0
        lax.fori_loop(0, _CHUNK // _LANES, fix_body, 0)

        pltpu.sync_copy(rows_v, out_hbm.at[pl.ds(base, _CHUNK)])
        return 0

    lax.fori_loop(0, n_chunks, chunk_body, 0)


def kernel(input_ids, old_table, new_table):
    batch, seq = input_ids.shape
    n_ids = batch * seq
    flat_ids = input_ids.reshape(n_ids)

    mesh = plsc.VectorSubcoreMesh(core_axis_name="c", subcore_axis_name="s")
    run = pl.kernel(
        _body,
        out_type=jax.ShapeDtypeStruct((n_ids, _EMBED_DIM), jnp.float32),
        mesh=mesh,
        scratch_types=[
            pltpu.VMEM((_NEW_VOCAB, _EMBED_DIM), jnp.float32),
            pltpu.VMEM((_CHUNK,), jnp.int32),
            pltpu.VMEM((_CHUNK,), jnp.int32),
            pltpu.VMEM((_CHUNK, _EMBED_DIM), jnp.float32),
            pltpu.SemaphoreType.DMA,
        ],
    )
    out = run(flat_ids, old_table, new_table)
    return out.reshape(batch, seq, _EMBED_DIM)


# speedup vs baseline: 4.2543x; 4.2543x over previous
import jax
import jax.numpy as jnp
from jax import lax
from jax.experimental import pallas as pl
from jax.experimental.pallas import tpu as pltpu
from jax.experimental.pallas import tpu_sc as plsc

_OLD_VOCAB = 1000000
_NEW_VOCAB = 1000
_EMBED_DIM = 64

_NUM_WORKERS = 32
_CHUNK = 512
_SUB = 128
_LANES = 16


def _body(ids_hbm, old_hbm, new_hbm, out_hbm,
          newtab_v, raw_v, idx_v, rows_v, gsem):
    n_ids = ids_hbm.shape[0]
    per_w = n_ids // _NUM_WORKERS
    n_chunks = per_w // _CHUNK
    wid = lax.axis_index("s") * 2 + lax.axis_index("c")
    base_w = wid * per_w

    pltpu.sync_copy(new_hbm, newtab_v)

    iota16 = lax.iota(jnp.int32, _LANES)

    def chunk_body(g, _):
        base = base_w + g * _CHUNK
        pltpu.sync_copy(ids_hbm.at[pl.ds(base, _CHUNK)], raw_v)

        def clip_body(i, _):
            v = raw_v[pl.ds(i * _LANES, _LANES)]
            idx_v[pl.ds(i * _LANES, _LANES)] = jnp.minimum(v, _OLD_VOCAB - 1)
            return 0
        lax.fori_loop(0, _CHUNK // _LANES, clip_body, 0)

        copies = []
        for j in range(_CHUNK // _SUB):
            copies.append(pltpu.async_copy(
                old_hbm.at[idx_v.at[pl.ds(j * _SUB, _SUB)]],
                rows_v.at[pl.ds(j * _SUB, _SUB)], gsem))
        for c in copies:
            c.wait()

        def fix_body(i, _):
            v = raw_v[pl.ds(i * _LANES, _LANES)]
            m = v >= _OLD_VOCAB

            cnt = plsc.all_reduce_population_count(m)

            @pl.when(cnt[0] > 0)
            def _():
                nid = jnp.maximum(v - _OLD_VOCAB, 0)
                rowpos = iota16 + i * _LANES

                def d_body(d, _):
                    dvec = jnp.full((_LANES,), d, jnp.int32)
                    vals = plsc.load_gather(newtab_v, [nid, dvec], mask=m)
                    plsc.store_scatter(rows_v, [rowpos, dvec], vals, mask=m)
                    return 0
                lax.fori_loop(0, _EMBED_DIM, d_body, 0)
            return 0
        lax.fori_loop(0, _CHUNK // _LANES, fix_body, 0)

        pltpu.sync_copy(rows_v, out_hbm.at[pl.ds(base, _CHUNK)])
        return 0

    lax.fori_loop(0, n_chunks, chunk_body, 0)


def kernel(input_ids, old_table, new_table):
    batch, seq = input_ids.shape
    n_ids = batch * seq
    flat_ids = input_ids.reshape(n_ids)

    mesh = plsc.VectorSubcoreMesh(core_axis_name="c", subcore_axis_name="s")
    run = pl.kernel(
        _body,
        out_type=jax.ShapeDtypeStruct((n_ids, _EMBED_DIM), jnp.float32),
        mesh=mesh,
        compiler_params=pltpu.CompilerParams(
            needs_layout_passes=False, use_tc_tiling_on_sc=False),
        scratch_types=[
            pltpu.VMEM((_NEW_VOCAB, _EMBED_DIM), jnp.float32),
            pltpu.VMEM((_CHUNK,), jnp.int32),
            pltpu.VMEM((_CHUNK,), jnp.int32),
            pltpu.VMEM((_CHUNK, _EMBED_DIM), jnp.float32),
            pltpu.SemaphoreType.DMA,
        ],
    )
    out = run(flat_ids, old_table, new_table)
    return out.reshape(batch, seq, _EMBED_DIM)


# speedup vs baseline: 4.7474x; 1.1159x over previous
import jax
import jax.numpy as jnp
from jax import lax
from jax.experimental import pallas as pl
from jax.experimental.pallas import tpu as pltpu
from jax.experimental.pallas import tpu_sc as plsc

_OLD_VOCAB = 1000000
_NEW_VOCAB = 1000
_EMBED_DIM = 64

_NUM_WORKERS = 32
_CHUNK = 256
_SUB = 128
_LANES = 16
_NBUF = 2


def _body(ids_hbm, old_hbm, new_hbm, out_hbm,
          newtab_v, raw_v, idx_v, rows_v, gsems, wsems):
    n_ids = ids_hbm.shape[0]
    per_w = n_ids // _NUM_WORKERS
    n_chunks = per_w // _CHUNK
    n_steps = n_chunks // _NBUF
    wid = lax.axis_index("s") * 2 + lax.axis_index("c")
    base_w = wid * per_w

    pltpu.sync_copy(new_hbm, newtab_v)

    iota16 = lax.iota(jnp.int32, _LANES)

    def raw(b):
        return raw_v.at[pl.ds(b * _CHUNK, _CHUNK)]

    def idx(b):
        return idx_v.at[pl.ds(b * _CHUNK, _CHUNK)]

    def rows(b):
        return rows_v.at[pl.ds(b * _CHUNK, _CHUNK)]

    def load_and_clip(c, b):
        base = base_w + c * _CHUNK
        pltpu.sync_copy(ids_hbm.at[pl.ds(base, _CHUNK)], raw(b))

        def clip_body(i, _):
            v = raw_v[pl.ds(b * _CHUNK + i * _LANES, _LANES)]
            idx_v[pl.ds(b * _CHUNK + i * _LANES, _LANES)] = (
                jnp.minimum(v, _OLD_VOCAB - 1))
            return 0
        lax.fori_loop(0, _CHUNK // _LANES, clip_body, 0)

    def fire_gathers(b):
        for j in range(_CHUNK // _SUB):
            pltpu.async_copy(
                old_hbm.at[idx(b).at[pl.ds(j * _SUB, _SUB)]],
                rows(b).at[pl.ds(j * _SUB, _SUB)], gsems[b])

    def wait_gathers(b):
        for j in range(_CHUNK // _SUB):
            pltpu.make_async_copy(
                old_hbm.at[idx(b).at[pl.ds(j * _SUB, _SUB)]],
                rows(b).at[pl.ds(j * _SUB, _SUB)], gsems[b]).wait()

    def fixup(b):
        def fix_body(i, _):
            v = raw_v[pl.ds(b * _CHUNK + i * _LANES, _LANES)]
            m = v >= _OLD_VOCAB
            cnt = plsc.all_reduce_population_count(m)

            @pl.when(cnt[0] > 0)
            def _():
                nid = jnp.maximum(v - _OLD_VOCAB, 0)
                rowpos = iota16 + (b * _CHUNK + i * _LANES)

                def d_body(d, _):
                    dvec = jnp.full((_LANES,), d, jnp.int32)
                    vals = plsc.load_gather(newtab_v, [nid, dvec], mask=m)
                    plsc.store_scatter(rows_v, [rowpos, dvec], vals, mask=m)
                    return 0
                lax.fori_loop(0, _EMBED_DIM, d_body, 0)
            return 0
        lax.fori_loop(0, _CHUNK // _LANES, fix_body, 0)

    for b in range(_NBUF):
        load_and_clip(b, b)
        fire_gathers(b)

    def step(g, _):
        for b in range(_NBUF):
            c = g * _NBUF + b
            wait_gathers(b)
            fixup(b)
            wdesc = pltpu.make_async_copy(
                rows(b), out_hbm.at[pl.ds(base_w + c * _CHUNK, _CHUNK)],
                wsems[b])
            wdesc.start()

            @pl.when(g < n_steps - 1)
            def _():
                load_and_clip(c + _NBUF, b)
            wdesc.wait()

            @pl.when(g < n_steps - 1)
            def _():
                fire_gathers(b)
        return 0

    lax.fori_loop(0, n_steps, step, 0)


def kernel(input_ids, old_table, new_table):
    batch, seq = input_ids.shape
    n_ids = batch * seq
    flat_ids = input_ids.reshape(n_ids)

    mesh = plsc.VectorSubcoreMesh(core_axis_name="c", subcore_axis_name="s")
    run = pl.kernel(
        _body,
        out_type=jax.ShapeDtypeStruct((n_ids, _EMBED_DIM), jnp.float32),
        mesh=mesh,
        compiler_params=pltpu.CompilerParams(
            needs_layout_passes=False, use_tc_tiling_on_sc=False),
        scratch_types=[
            pltpu.VMEM((_NEW_VOCAB, _EMBED_DIM), jnp.float32),
            pltpu.VMEM((_NBUF * _CHUNK,), jnp.int32),
            pltpu.VMEM((_NBUF * _CHUNK,), jnp.int32),
            pltpu.VMEM((_NBUF * _CHUNK, _EMBED_DIM), jnp.float32),
            [pltpu.SemaphoreType.DMA] * _NBUF,
            [pltpu.SemaphoreType.DMA] * _NBUF,
        ],
    )
    out = run(flat_ids, old_table, new_table)
    return out.reshape(batch, seq, _EMBED_DIM)
